# baseline (speedup 1.0000x reference)
import functools

import jax
import jax.numpy as jnp
from jax import lax
from jax.experimental import pallas as pl
from jax.experimental.pallas import tpu as pltpu
from jax.experimental.pallas import tpu_sc as plsc

_B, _S, _F = 16384, 200, 9
_NW = 32
_PB = _B // _NW
_CHUNK = 64
_V27 = 27
_LANES = 128

_GROUPS = [0, 16, 32, 48, 64, 80, 96, 112, 128, 144, 160, 176, 184]


def _sc_body(idx_hbm, w_hbm, out_hbm, idx_v, lut_v, row_a, row_b, sem_a, sem_b):
    wid = lax.axis_index("s") * 2 + lax.axis_index("c")
    base = wid * _PB

    for v in range(_V27):
        pltpu.sync_copy(
            w_hbm.at[pl.ds(v, 1)], lut_v.at[pl.ds(v, 1)]
        )

    iota = lax.iota(jnp.int32, 16)

    def build_row(local_b, row_ref):
        for s0 in _GROUPS:
            idx16 = idx_v[local_b, pl.ds(s0, 16)]
            idxc = jnp.minimum(jnp.maximum(idx16, 0), _V27 - 1)
            s16 = iota + s0
            for j in range(_F):
                jv = jnp.full((16,), j, jnp.int32)
                val = plsc.load_gather(lut_v, [idxc, jv])
                plsc.store_scatter(row_ref, [s16, jv], val)

    def step(b, carry):
        @pl.when(lax.rem(b, _CHUNK) == 0)
        def _stage():
            start = pl.multiple_of(base + b, 8)
            pltpu.sync_copy(
                idx_hbm.at[pl.ds(start, _CHUNK)], idx_v
            )

        local_b = lax.rem(b, _CHUNK)
        gb = base + b
        parity = lax.rem(b, 2)

        @pl.when(parity == 0)
        def _even():
            @pl.when(b >= 2)
            def _wait():
                pltpu.make_async_copy(
                    out_hbm.at[gb], row_a, sem_a
                ).wait()
            build_row(local_b, row_a)
            pltpu.make_async_copy(
                row_a, out_hbm.at[gb], sem_a
            ).start()

        @pl.when(parity == 1)
        def _odd():
            @pl.when(b >= 2)
            def _wait():
                pltpu.make_async_copy(
                    out_hbm.at[gb], row_b, sem_b
                ).wait()
            build_row(local_b, row_b)
            pltpu.make_async_copy(
                row_b, out_hbm.at[gb], sem_b
            ).start()

        return carry

    lax.fori_loop(0, _PB, step, 0)
    pltpu.make_async_copy(out_hbm.at[base], row_a, sem_a).wait()
    pltpu.make_async_copy(out_hbm.at[base], row_b, sem_b).wait()


@functools.partial(jax.jit, static_argnames=())
def kernel(input, weight):
    mesh = plsc.VectorSubcoreMesh(
        core_axis_name="c", subcore_axis_name="s", num_cores=2, num_subcores=16
    )
    sc = pl.kernel(
        _sc_body,
        out_type=jax.ShapeDtypeStruct((_B, _S, _F), jnp.float32),
        mesh=mesh,
        scratch_types=[
            pltpu.VMEM((_CHUNK, _S), jnp.int32),
            pltpu.VMEM((_V27, _F), jnp.float32),
            pltpu.VMEM((_S, _F), jnp.float32),
            pltpu.VMEM((_S, _F), jnp.float32),
            pltpu.SemaphoreType.DMA,
            pltpu.SemaphoreType.DMA,
        ],
        compiler_params=pltpu.CompilerParams(needs_layout_passes=False),
    )
    return sc(input, weight)
